# speedup vs baseline: 10.9121x; 10.9121x over previous
import functools

import jax
import jax.numpy as jnp
from jax import lax
from jax.experimental import pallas as pl
from jax.experimental.pallas import tpu as pltpu
from jax.experimental.pallas import tpu_sc as plsc

N = 10000
E = 320000
D = 128

NC = 2
NS = 16
NW = NC * NS

EB = 128
N_PAD = 10240
E_PAD = 323584
EPT = E_PAD // NW
NB = EPT // EB
RPT = N_PAD // NS

_MESH = plsc.VectorSubcoreMesh(core_axis_name="c", subcore_axis_name="s")


def _deg_body(dst_hbm, z16_hbm, out0, out1, idxb, ones, acc, sem):
    c = lax.axis_index("c")
    s = lax.axis_index("s")
    wid = s * NC + c
    r0 = s * RPT

    pltpu.async_copy(z16_hbm.at[pl.ds(r0, RPT)], acc.at[pl.ds(r0, RPT)], sem).wait()

    def fill(i, carry):
        ones[i, :] = jnp.ones((16,), jnp.float32)
        return carry

    lax.fori_loop(0, EB, fill, 0)
    plsc.subcore_barrier()

    def body(i, carry):
        base = wid * EPT + i * EB
        pltpu.sync_copy(dst_hbm.at[pl.ds(base, EB)], idxb)
        pltpu.sync_copy(ones, acc.at[idxb], add=True)
        return carry

    lax.fori_loop(0, NB, body, 0)
    plsc.subcore_barrier()

    @pl.when(c == 0)
    def _():
        pltpu.async_copy(acc.at[pl.ds(r0, RPT)], out0.at[pl.ds(r0, RPT)], sem).wait()

    @pl.when(c == 1)
    def _():
        pltpu.async_copy(acc.at[pl.ds(r0, RPT)], out1.at[pl.ds(r0, RPT)], sem).wait()


_deg_call = pl.kernel(
    _deg_body,
    out_type=(
        jax.ShapeDtypeStruct((N_PAD, 16), jnp.float32),
        jax.ShapeDtypeStruct((N_PAD, 16), jnp.float32),
    ),
    mesh=_MESH,
    scratch_types=[
        pltpu.VMEM((EB,), jnp.int32),
        pltpu.VMEM((EB, 16), jnp.float32),
        pltpu.VMEM_SHARED((N_PAD, 16), jnp.float32),
        pltpu.SemaphoreType.DMA,
    ],
)


def _gcn_body(y_hbm, z_hbm, src_hbm, dst_hbm, out0, out1, srcb, dstb, rows, acc, sem):
    c = lax.axis_index("c")
    s = lax.axis_index("s")
    wid = s * NC + c
    r0 = s * RPT

    @pl.when(c == 0)
    def _():
        pltpu.async_copy(y_hbm.at[pl.ds(r0, RPT)], acc.at[pl.ds(r0, RPT)], sem).wait()

    @pl.when(c == 1)
    def _():
        pltpu.async_copy(z_hbm.at[pl.ds(r0, RPT)], acc.at[pl.ds(r0, RPT)], sem).wait()

    plsc.subcore_barrier()

    def body(i, carry):
        base = wid * EPT + i * EB
        pltpu.sync_copy(src_hbm.at[pl.ds(base, EB)], srcb)
        pltpu.sync_copy(dst_hbm.at[pl.ds(base, EB)], dstb)
        pltpu.async_copy(y_hbm.at[srcb], rows, sem).wait()
        pltpu.sync_copy(rows, acc.at[dstb], add=True)
        return carry

    lax.fori_loop(0, NB, body, 0)
    plsc.subcore_barrier()

    @pl.when(c == 0)
    def _():
        pltpu.async_copy(acc.at[pl.ds(r0, RPT)], out0.at[pl.ds(r0, RPT)], sem).wait()

    @pl.when(c == 1)
    def _():
        pltpu.async_copy(acc.at[pl.ds(r0, RPT)], out1.at[pl.ds(r0, RPT)], sem).wait()


_gcn_call = pl.kernel(
    _gcn_body,
    out_type=(
        jax.ShapeDtypeStruct((N_PAD, D), jnp.float32),
        jax.ShapeDtypeStruct((N_PAD, D), jnp.float32),
    ),
    mesh=_MESH,
    scratch_types=[
        pltpu.VMEM((EB,), jnp.int32),
        pltpu.VMEM((EB,), jnp.int32),
        pltpu.VMEM((EB, D), jnp.float32),
        pltpu.VMEM_SHARED((N_PAD, D), jnp.float32),
        pltpu.SemaphoreType.DMA,
    ],
)


BN = 640
_GRID = (N_PAD // BN,)
_row_spec = pl.BlockSpec((BN, D), lambda i: (i, 0))
_deg_spec = pl.BlockSpec((BN, 16), lambda i: (i, 0))
_w_spec = pl.BlockSpec((D, D), lambda i: (0, 0))
_b_spec = pl.BlockSpec((1, D), lambda i: (0, 0))
_f32 = functools.partial(jax.ShapeDtypeStruct, dtype=jnp.float32)


def _dinv(d0_ref, d1_ref):
    return lax.rsqrt(d0_ref[:, :1] + d1_ref[:, :1] + 1.0)


def _mm_body(x_ref, w_ref, o_ref):
    o_ref[:, :] = jnp.dot(x_ref[:, :], w_ref[:, :], preferred_element_type=jnp.float32)


_mm_call = pl.pallas_call(
    _mm_body,
    grid=_GRID,
    in_specs=[_row_spec, _w_spec],
    out_specs=_row_spec,
    out_shape=_f32((N_PAD, D)),
)


def _scale_body(xw_ref, d0_ref, d1_ref, o_ref):
    o_ref[:, :] = xw_ref[:, :] * _dinv(d0_ref, d1_ref)


_scale_call = pl.pallas_call(
    _scale_body,
    grid=_GRID,
    in_specs=[_row_spec, _deg_spec, _deg_spec],
    out_specs=_row_spec,
    out_shape=_f32((N_PAD, D)),
)


def _mid_body(a0_ref, a1_ref, d0_ref, d1_ref, b1_ref, w2_ref, o_ref):
    dinv = _dinv(d0_ref, d1_ref)
    h = jnp.maximum((a0_ref[:, :] + a1_ref[:, :]) * dinv + b1_ref[:, :], 0.0)
    o_ref[:, :] = jnp.dot(h, w2_ref[:, :], preferred_element_type=jnp.float32) * dinv


_mid_call = pl.pallas_call(
    _mid_body,
    grid=_GRID,
    in_specs=[_row_spec, _row_spec, _deg_spec, _deg_spec, _b_spec, _w_spec],
    out_specs=_row_spec,
    out_shape=_f32((N_PAD, D)),
)


def _fin_body(a0_ref, a1_ref, d0_ref, d1_ref, b2_ref, o_ref):
    o_ref[:, :] = (a0_ref[:, :] + a1_ref[:, :]) * _dinv(d0_ref, d1_ref) + b2_ref[:, :]


_fin_call = pl.pallas_call(
    _fin_body,
    grid=_GRID,
    in_specs=[_row_spec, _row_spec, _deg_spec, _deg_spec, _b_spec],
    out_specs=_row_spec,
    out_shape=_f32((N_PAD, D)),
)


def kernel(x, edge_index, W1, b1, W2, b2):
    src = edge_index[0].astype(jnp.int32)
    dst = edge_index[1].astype(jnp.int32)
    npad_e = E_PAD - E
    pad_src = jnp.full((npad_e,), N, jnp.int32)
    pad_dst = N + (jnp.arange(npad_e, dtype=jnp.int32) % (N_PAD - N))
    src_p = jnp.concatenate([src, pad_src])
    dst_p = jnp.concatenate([dst, pad_dst])

    x_p = jnp.zeros((N_PAD, D), jnp.float32).at[:N].set(x)
    z2 = jnp.zeros((N_PAD, D), jnp.float32)
    z16 = jnp.zeros((N_PAD, 16), jnp.float32)

    d0, d1 = _deg_call(dst_p, z16)
    xw1 = _mm_call(x_p, W1)
    y1 = _scale_call(xw1, d0, d1)
    a0, a1 = _gcn_call(y1, z2, src_p, dst_p)
    y2 = _mid_call(a0, a1, d0, d1, b1.reshape(1, D), W2)
    a0b, a1b = _gcn_call(y2, z2, src_p, dst_p)
    out = _fin_call(a0b, a1b, d0, d1, b2.reshape(1, D))
    return out[:N]
